# Initial kernel scaffold; baseline (speedup 1.0000x reference)
import jax
import jax.numpy as jnp
from jax.experimental import pallas as pl


def kernel(x, centers):
    raise NotImplementedError("write your pallas kernel here")


# speedup vs baseline: 1.0199x; 1.0199x over previous
import jax
import jax.numpy as jnp
from jax.experimental import pallas as pl
from jax.experimental.pallas import tpu as pltpu

_TB = 256
_K = 8192
_D = 256


def _assign_kernel(x_ref, c_ref, out_ref):
    x = x_ref[...]
    c = c_ref[...]
    xn = jnp.sum(x * x, axis=1, keepdims=True)
    cn = jnp.sum(c * c, axis=1)
    prod = jax.lax.dot_general(
        x, c, (((1,), (1,)), ((), ())),
        preferred_element_type=jnp.float32)
    dist = jnp.abs(xn - 2.0 * prod + cn[None, :])
    m = jnp.min(dist, axis=1, keepdims=True)
    idx = jax.lax.broadcasted_iota(jnp.int32, dist.shape, 1)
    amin = jnp.min(jnp.where(dist == m, idx, jnp.int32(_K)), axis=1)
    out_ref[...] = amin.reshape(1, 1, _TB)


def kernel(x, centers):
    b, t, d = x.shape
    n = b * t
    nblocks = n // _TB
    x2 = x.reshape(n, d)
    out = pl.pallas_call(
        _assign_kernel,
        grid=(nblocks,),
        in_specs=[
            pl.BlockSpec((_TB, _D), lambda i: (i, 0)),
            pl.BlockSpec((_K, _D), lambda i: (0, 0)),
        ],
        out_specs=pl.BlockSpec((1, 1, _TB), lambda i: (i, 0, 0)),
        out_shape=jax.ShapeDtypeStruct((nblocks, 1, _TB), jnp.int32),
        compiler_params=pltpu.CompilerParams(
            dimension_semantics=("arbitrary",),
        ),
    )(x2, centers)
    return out.reshape(b, t)


# speedup vs baseline: 1.6208x; 1.5891x over previous
import jax
import jax.numpy as jnp
from jax.experimental import pallas as pl
from jax.experimental.pallas import tpu as pltpu

_TB = 256
_K = 8192
_D = 256


def _assign_kernel(x_ref, c_ref, out_ref, cn_ref):
    @pl.when(pl.program_id(0) == 0)
    def _():
        c = c_ref[...]
        cn_ref[...] = jnp.sum(c * c, axis=1).reshape(1, _K)

    xm2 = x_ref[...] * -2.0
    prod = jax.lax.dot_general(
        xm2, c_ref[...], (((1,), (1,)), ((), ())),
        preferred_element_type=jnp.float32)
    val = prod + cn_ref[...]
    idx = jax.lax.broadcasted_iota(jnp.int32, val.shape, 1)
    width = _K
    while width > 128:
        half = width // 2
        v0, v1 = val[:, :half], val[:, half:]
        i0, i1 = idx[:, :half], idx[:, half:]
        mask = v1 < v0
        val = jnp.where(mask, v1, v0)
        idx = jnp.where(mask, i1, i0)
        width = half
    m = jnp.min(val, axis=1, keepdims=True)
    amin = jnp.min(jnp.where(val == m, idx, jnp.int32(_K)), axis=1)
    out_ref[...] = amin.reshape(1, 1, _TB)


def kernel(x, centers):
    b, t, d = x.shape
    n = b * t
    nblocks = n // _TB
    x2 = x.reshape(n, d)
    out = pl.pallas_call(
        _assign_kernel,
        grid=(nblocks,),
        in_specs=[
            pl.BlockSpec((_TB, _D), lambda i: (i, 0)),
            pl.BlockSpec((_K, _D), lambda i: (0, 0)),
        ],
        out_specs=pl.BlockSpec((1, 1, _TB), lambda i: (i, 0, 0)),
        out_shape=jax.ShapeDtypeStruct((nblocks, 1, _TB), jnp.int32),
        scratch_shapes=[pltpu.VMEM((1, _K), jnp.float32)],
        compiler_params=pltpu.CompilerParams(
            dimension_semantics=("arbitrary",),
        ),
    )(x2, centers)
    return out.reshape(b, t)


# speedup vs baseline: 1.7273x; 1.0657x over previous
import jax
import jax.numpy as jnp
from jax.experimental import pallas as pl
from jax.experimental.pallas import tpu as pltpu

_TB = 512
_K = 8192
_D = 256


def _assign_kernel(x_ref, c_ref, out_ref, cn_ref):
    @pl.when(pl.program_id(0) == 0)
    def _():
        c = c_ref[...]
        cn_ref[...] = jnp.sum(c * c, axis=1).reshape(1, _K)

    xm2 = x_ref[...] * -2.0
    prod = jax.lax.dot_general(
        xm2, c_ref[...], (((1,), (1,)), ((), ())),
        preferred_element_type=jnp.float32)
    val = prod + cn_ref[...]
    idx = jax.lax.broadcasted_iota(jnp.int32, val.shape, 1)
    width = _K
    while width > 128:
        half = width // 2
        v0, v1 = val[:, :half], val[:, half:]
        i0, i1 = idx[:, :half], idx[:, half:]
        mask = v1 < v0
        val = jnp.where(mask, v1, v0)
        idx = jnp.where(mask, i1, i0)
        width = half
    m = jnp.min(val, axis=1, keepdims=True)
    amin = jnp.min(jnp.where(val == m, idx, jnp.int32(_K)), axis=1)
    out_ref[...] = amin.reshape(1, 1, _TB)


def kernel(x, centers):
    b, t, d = x.shape
    n = b * t
    nblocks = n // _TB
    x2 = x.reshape(n, d)
    out = pl.pallas_call(
        _assign_kernel,
        grid=(nblocks,),
        in_specs=[
            pl.BlockSpec((_TB, _D), lambda i: (i, 0)),
            pl.BlockSpec((_K, _D), lambda i: (0, 0)),
        ],
        out_specs=pl.BlockSpec((1, 1, _TB), lambda i: (i, 0, 0)),
        out_shape=jax.ShapeDtypeStruct((nblocks, 1, _TB), jnp.int32),
        scratch_shapes=[pltpu.VMEM((1, _K), jnp.float32)],
        compiler_params=pltpu.CompilerParams(
            dimension_semantics=("arbitrary",),
        ),
    )(x2, centers)
    return out.reshape(b, t)


# speedup vs baseline: 1.8539x; 1.0733x over previous
import jax
import jax.numpy as jnp
from jax.experimental import pallas as pl
from jax.experimental.pallas import tpu as pltpu

_TB = 512
_K = 8192
_D = 256


def _assign_kernel(x_ref, c_ref, out_ref, cn_ref):
    @pl.when(pl.program_id(0) == 0)
    def _():
        c = c_ref[...]
        cn_ref[...] = jnp.sum(c * c, axis=1).reshape(1, _K)

    xm2 = x_ref[...] * -2.0
    prod = jax.lax.dot_general(
        xm2, c_ref[...], (((1,), (1,)), ((), ())),
        preferred_element_type=jnp.float32)
    cn = cn_ref[...]
    rv, ri = None, None
    ch = 1024
    for j in range(_K // ch):
        pc = prod[:, j * ch:(j + 1) * ch] + cn[:, j * ch:(j + 1) * ch]
        v0, v1 = pc[:, :512], pc[:, 512:]
        mask = v1 < v0
        cv = jnp.where(mask, v1, v0)
        co = jnp.where(mask, 512.0, 0.0)
        for h in (256, 128):
            v0, v1 = cv[:, :h], cv[:, h:]
            o0, o1 = co[:, :h], co[:, h:]
            mask = v1 < v0
            cv = jnp.where(mask, v1, v0)
            co = jnp.where(mask, o1 + float(h), o0)
        if rv is None:
            rv, ri = cv, co
        else:
            mask = cv < rv
            rv = jnp.where(mask, cv, rv)
            ri = jnp.where(mask, co + float(j * ch), ri)
    lane = jax.lax.broadcasted_iota(jnp.float32, ri.shape, 1)
    full = ri + lane
    m = jnp.min(rv, axis=1, keepdims=True)
    amin = jnp.min(jnp.where(rv == m, full, float(_K)), axis=1)
    out_ref[...] = amin.astype(jnp.int32).reshape(1, 1, _TB)


def kernel(x, centers):
    b, t, d = x.shape
    n = b * t
    nblocks = n // _TB
    x2 = x.reshape(n, d)
    out = pl.pallas_call(
        _assign_kernel,
        grid=(nblocks,),
        in_specs=[
            pl.BlockSpec((_TB, _D), lambda i: (i, 0)),
            pl.BlockSpec((_K, _D), lambda i: (0, 0)),
        ],
        out_specs=pl.BlockSpec((1, 1, _TB), lambda i: (i, 0, 0)),
        out_shape=jax.ShapeDtypeStruct((nblocks, 1, _TB), jnp.int32),
        scratch_shapes=[pltpu.VMEM((1, _K), jnp.float32)],
        compiler_params=pltpu.CompilerParams(
            dimension_semantics=("arbitrary",),
        ),
    )(x2, centers)
    return out.reshape(b, t)


# speedup vs baseline: 2.0415x; 1.1011x over previous
import jax
import jax.numpy as jnp
from jax.experimental import pallas as pl
from jax.experimental.pallas import tpu as pltpu

_TB = 1024
_K = 8192
_D = 256


def _assign_kernel(x_ref, c_ref, out_ref, cn_ref):
    @pl.when(pl.program_id(0) == 0)
    def _():
        c = c_ref[...]
        cn_ref[...] = jnp.sum(c * c, axis=1).reshape(1, _K)

    xm2 = x_ref[...] * -2.0
    prod = jax.lax.dot_general(
        xm2, c_ref[...], (((1,), (1,)), ((), ())),
        preferred_element_type=jnp.float32)
    cn = cn_ref[...]
    rv, ri = None, None
    ch = 1024
    for j in range(_K // ch):
        pc = prod[:, j * ch:(j + 1) * ch] + cn[:, j * ch:(j + 1) * ch]
        v0, v1 = pc[:, :512], pc[:, 512:]
        mask = v1 < v0
        cv = jnp.where(mask, v1, v0)
        co = jnp.where(mask, 512.0, 0.0)
        for h in (256, 128):
            v0, v1 = cv[:, :h], cv[:, h:]
            o0, o1 = co[:, :h], co[:, h:]
            mask = v1 < v0
            cv = jnp.where(mask, v1, v0)
            co = jnp.where(mask, o1 + float(h), o0)
        if rv is None:
            rv, ri = cv, co
        else:
            mask = cv < rv
            rv = jnp.where(mask, cv, rv)
            ri = jnp.where(mask, co + float(j * ch), ri)
    lane = jax.lax.broadcasted_iota(jnp.int32, ri.shape, 1).astype(jnp.float32)
    full = ri + lane
    m = jnp.min(rv, axis=1, keepdims=True)
    amin = jnp.min(jnp.where(rv == m, full, float(_K)), axis=1)
    out_ref[...] = amin.astype(jnp.int32).reshape(1, 1, _TB)


def kernel(x, centers):
    b, t, d = x.shape
    n = b * t
    nblocks = n // _TB
    x2 = x.reshape(n, d)
    out = pl.pallas_call(
        _assign_kernel,
        grid=(nblocks,),
        in_specs=[
            pl.BlockSpec((_TB, _D), lambda i: (i, 0)),
            pl.BlockSpec((_K, _D), lambda i: (0, 0)),
        ],
        out_specs=pl.BlockSpec((1, 1, _TB), lambda i: (i, 0, 0)),
        out_shape=jax.ShapeDtypeStruct((nblocks, 1, _TB), jnp.int32),
        scratch_shapes=[pltpu.VMEM((1, _K), jnp.float32)],
        compiler_params=pltpu.CompilerParams(
            dimension_semantics=("arbitrary",),
        ),
    )(x2, centers)
    return out.reshape(b, t)
